# baseline (speedup 1.0000x reference)
import jax
import jax.numpy as jnp
import numpy as np
from jax import lax
from jax.experimental import pallas as pl
from jax.experimental.pallas import tpu as pltpu
from jax.experimental.pallas import tpu_sc as plsc

VOCAB = 1000000
D = 64
SEQ = 200
BATCH = 1024
NTOK = BATCH * SEQ
NC, NS, LANES = 2, 16, 16
NW = NC * NS
SCALE = float(D) ** 0.5
PITCH = D

NBLK_FULL = VOCAB // 128
TAIL_I = VOCAB - NBLK_FULL * 128
BLK_PER_W = NBLK_FULL // NW
BLK_REM = NBLK_FULL - BLK_PER_W * NW

NU = SEQ * (BATCH // 128)
U_PER_W = NU // NW
NBUF = 2
ANBUF = 4


def _pos_encoding_flat() -> np.ndarray:
    positions = np.arange(SEQ).reshape(-1, 1)
    dims = np.arange(D // 2).reshape(1, -1)
    angles = positions / np.power(10000, 2 * dims / D)
    pe = np.zeros((SEQ, D))
    pe[:, 0::2] = np.sin(angles)
    pe[:, 1::2] = np.cos(angles)
    return pe.astype(np.float32).reshape(-1)


_PE_FLAT = _pos_encoding_flat()


def _transpose_block(vbuf, obuf, nrows):
    ivec = [lax.iota(jnp.int32, LANES) + iv * LANES
            for iv in range(nrows // LANES)]
    iv_idx = [ivec[iv] * D for iv in range(nrows // LANES)]

    @plsc.parallel_loop(0, D, 1, unroll=8)
    def col(c):
        cc = jnp.full((LANES,), c, jnp.int32)
        for iv in range(nrows // LANES):
            v = vbuf[c, pl.ds(iv * LANES, LANES)]
            rot = (cc + ivec[iv]) & (D - 1)
            plsc.store_scatter(obuf, [iv_idx[iv] + rot], v * SCALE)


def _phase_a_body(tableT, tail_flat, tp_out, vbufs, obufs, gsems, ssems):
    wid = lax.axis_index("s") * NC + lax.axis_index("c")
    start = wid * BLK_PER_W + jnp.minimum(wid, BLK_REM)
    nblk = BLK_PER_W + jnp.where(wid < BLK_REM, 1, 0)

    def start_in(blk, slot):
        pltpu.async_copy(
            tableT.at[:, pl.ds(blk * 128, 128)], vbufs[slot], gsems[slot])

    def wait_in(slot):
        pltpu.make_async_copy(
            tableT.at[:, pl.ds(0, 128)], vbufs[slot], gsems[slot]).wait()

    def start_out(blk, slot):
        pltpu.async_copy(
            obufs[slot], tp_out.at[pl.ds(blk * 128 * PITCH, 128 * PITCH)],
            ssems[slot])

    def wait_out(slot):
        pltpu.make_async_copy(
            obufs[slot], tp_out.at[pl.ds(0, 128 * PITCH)], ssems[slot]).wait()

    start_in(start, 0)

    def step(jo, carry):
        for b in range(ANBUF):
            j = jo * ANBUF + b
            nb = (b + 1) % ANBUF

            @pl.when(j < nblk)
            def _():
                @pl.when(j + 1 < nblk)
                def _():
                    @pl.when(j + 1 >= ANBUF)
                    def _():
                        wait_out(nb)
                    start_in(start + j + 1, nb)

                wait_in(b)
                _transpose_block(vbufs[b], obufs[b], 128)
                start_out(start + j, b)
        return carry

    nouter = (BLK_PER_W + 1 + ANBUF - 1) // ANBUF
    lax.fori_loop(0, nouter, step, 0)
    for b in range(ANBUF):
        wait_out(b)

    @pl.when(wid == NW - 1)
    def _():
        pltpu.sync_copy(tail_flat, obufs[1].at[pl.ds(0, TAIL_I * D)])

        kvec = [lax.iota(jnp.int32, LANES) + k * LANES
                for k in range(D // LANES)]

        def srow(i, carry):
            ii = jnp.full((LANES,), i, jnp.int32)
            for k in range(D // LANES):
                v = obufs[1][pl.ds(i * D + k * LANES, LANES)]
                rot = (kvec[k] + ii) & (D - 1)
                plsc.store_scatter(obufs[0], [ii * D + rot], v * SCALE)
            return carry

        lax.fori_loop(0, TAIL_I, srow, 0, unroll=2)
        pltpu.sync_copy(
            obufs[0].at[pl.ds(0, TAIL_I * PITCH)],
            tp_out.at[pl.ds(NBLK_FULL * 128 * PITCH, TAIL_I * PITCH)])


def _phase_b_body(xt2, tp, pe_hbm, out5, idxs, bufs, obufs, pe_v,
                  isems, gsems, ssems):
    wid = lax.axis_index("s") * NC + lax.axis_index("c")
    u0 = wid * U_PER_W

    pltpu.sync_copy(pe_hbm, pe_v)

    def start_idx(u, slot):
        pltpu.async_copy(xt2.at[u], idxs[slot], isems[slot])

    def wait_idx(slot):
        pltpu.make_async_copy(xt2.at[0], idxs[slot], isems[slot]).wait()

    def start_gather(slot):
        pltpu.async_copy(tp.at[idxs[slot]], bufs[slot], gsems[slot])

    def wait_gather(slot):
        pltpu.make_async_copy(tp.at[idxs[slot]], bufs[slot], gsems[slot]).wait()

    def start_out(u, slot):
        s = u // 8
        bb = lax.rem(u, 8)
        for cb in range(8):
            pltpu.async_copy(obufs[slot].at[cb], out5.at[s, cb, bb], ssems[slot])

    def wait_out(slot):
        for cb in range(8):
            pltpu.make_async_copy(
                obufs[slot].at[cb], out5.at[0, cb, 0], ssems[slot]).wait()

    def compute(u, slot):
        s = u // 8
        buf = bufs[slot]
        obuf = obufs[slot]
        idx_v = idxs[slot]
        bidx = [lax.iota(jnp.int32, LANES) + 16 * t for t in range(8)]
        vmod = [idx_v[pl.ds(16 * t, LANES)] & (D - 1) for t in range(8)]

        @plsc.parallel_loop(0, D, 1, unroll=8)
        def col(c):
            pev = plsc.load_gather(
                pe_v, [jnp.full((LANES,), s * D + c, jnp.int32)])
            cb = c // 8
            ci = lax.rem(c, 8)
            cc = jnp.full((LANES,), c, jnp.int32)
            for t in range(8):
                cols = (vmod[t] + cc) & (D - 1)
                v = plsc.load_gather(buf, [bidx[t], cols])
                obuf[cb, ci, pl.ds(t * LANES, LANES)] = v + pev

    start_idx(u0, 0)
    wait_idx(0)
    start_gather(0)
    start_idx(u0 + 1, 1)

    def step(jo, carry):
        for b in range(NBUF):
            j = jo * NBUF + b
            u = u0 + j
            nb = (b + 1) % NBUF

            @pl.when(j + 1 < U_PER_W)
            def _():
                wait_idx(nb)
                start_gather(nb)

            wait_gather(b)

            @pl.when(j >= NBUF)
            def _():
                wait_out(b)

            compute(u, b)
            start_out(u, b)

            @pl.when(j + 2 < U_PER_W)
            def _():
                start_idx(u + 2, b)
        return carry

    lax.fori_loop(0, U_PER_W // NBUF, step, 0)
    for b in range(NBUF):
        wait_out(b)


@jax.jit
def _run(x, table):
    mesh = plsc.VectorSubcoreMesh(core_axis_name="c", subcore_axis_name="s")

    tableT = table.T

    phase_a = pl.kernel(
        _phase_a_body,
        out_type=jax.ShapeDtypeStruct((VOCAB * PITCH,), jnp.float32),
        mesh=mesh,
        scratch_types=[
            [pltpu.VMEM((D, 128), jnp.float32) for _ in range(ANBUF)],
            [pltpu.VMEM((128 * PITCH,), jnp.float32) for _ in range(ANBUF)],
            [pltpu.SemaphoreType.DMA for _ in range(ANBUF)],
            [pltpu.SemaphoreType.DMA for _ in range(ANBUF)],
        ],
        compiler_params=pltpu.CompilerParams(use_tc_tiling_on_sc=True, needs_layout_passes=False),
    )
    tail_flat = jnp.reshape(table[NBLK_FULL * 128:, :], (TAIL_I * D,))
    tp_flat = phase_a(tableT, tail_flat)
    tp = jnp.reshape(tp_flat, (VOCAB, PITCH))

    xt2 = jnp.reshape(x.astype(jnp.int32).T, (NU, 128))
    pe_flat = jnp.asarray(_PE_FLAT)

    phase_b = pl.kernel(
        _phase_b_body,
        out_type=jax.ShapeDtypeStruct((SEQ, 8, 8, 8, 128), jnp.float32),
        mesh=mesh,
        scratch_types=[
            [pltpu.VMEM((128,), jnp.int32) for _ in range(NBUF)],
            [pltpu.VMEM((128, PITCH), jnp.float32) for _ in range(NBUF)],
            [pltpu.VMEM((8, 8, 128), jnp.float32) for _ in range(NBUF)],
            pltpu.VMEM((SEQ * D,), jnp.float32),
            [pltpu.SemaphoreType.DMA for _ in range(NBUF)],
            [pltpu.SemaphoreType.DMA for _ in range(NBUF)],
            [pltpu.SemaphoreType.DMA for _ in range(NBUF)],
        ],
        compiler_params=pltpu.CompilerParams(use_tc_tiling_on_sc=False, needs_layout_passes=False),
    )
    out5 = phase_b(xt2, tp, pe_flat)
    return jnp.transpose(out5, (2, 4, 0, 1, 3)).reshape(BATCH, SEQ, D)


def kernel(x, table):
    return _run(x, table)


# speedup vs baseline: 1.0239x; 1.0239x over previous
import jax
import jax.numpy as jnp
import numpy as np
from jax import lax
from jax.experimental import pallas as pl
from jax.experimental.pallas import tpu as pltpu
from jax.experimental.pallas import tpu_sc as plsc

VOCAB = 1000000
D = 64
SEQ = 200
BATCH = 1024
NTOK = BATCH * SEQ
NC, NS, LANES = 2, 16, 16
NW = NC * NS
SCALE = float(D) ** 0.5
PITCH = D

NBLK_FULL = VOCAB // 128
TAIL_I = VOCAB - NBLK_FULL * 128
BLK_PER_W = NBLK_FULL // NW
BLK_REM = NBLK_FULL - BLK_PER_W * NW

NU = SEQ * (BATCH // 128)
U_PER_W = NU // NW
NBUF = 2
ANBUF = 3


def _pos_encoding_flat() -> np.ndarray:
    positions = np.arange(SEQ).reshape(-1, 1)
    dims = np.arange(D // 2).reshape(1, -1)
    angles = positions / np.power(10000, 2 * dims / D)
    pe = np.zeros((SEQ, D))
    pe[:, 0::2] = np.sin(angles)
    pe[:, 1::2] = np.cos(angles)
    return pe.astype(np.float32).reshape(-1)


_PE_FLAT = _pos_encoding_flat()


def _transpose_block(vbuf, obuf, nrows):
    ivec = [lax.iota(jnp.int32, LANES) + iv * LANES
            for iv in range(nrows // LANES)]
    iv_idx = [ivec[iv] * D for iv in range(nrows // LANES)]

    @plsc.parallel_loop(0, D, 1, unroll=4)
    def col(c):
        cc = jnp.full((LANES,), c, jnp.int32)
        for iv in range(nrows // LANES):
            v = vbuf[c, pl.ds(iv * LANES, LANES)]
            rot = (cc + ivec[iv]) & (D - 1)
            plsc.store_scatter(obuf, [iv_idx[iv] + rot], v * SCALE)


def _phase_a_body(tableT, tail_flat, tp_out, vbufs, obufs, gsems, ssems):
    wid = lax.axis_index("s") * NC + lax.axis_index("c")
    start = wid * BLK_PER_W + jnp.minimum(wid, BLK_REM)
    nblk = BLK_PER_W + jnp.where(wid < BLK_REM, 1, 0)

    def start_in(blk, slot):
        pltpu.async_copy(
            tableT.at[:, pl.ds(blk * 128, 128)], vbufs[slot], gsems[slot])

    def wait_in(slot):
        pltpu.make_async_copy(
            tableT.at[:, pl.ds(0, 128)], vbufs[slot], gsems[slot]).wait()

    def start_out(blk, slot):
        pltpu.async_copy(
            obufs[slot], tp_out.at[pl.ds(blk * 128 * PITCH, 128 * PITCH)],
            ssems[slot])

    def wait_out(slot):
        pltpu.make_async_copy(
            obufs[slot], tp_out.at[pl.ds(0, 128 * PITCH)], ssems[slot]).wait()

    start_in(start, 0)

    def step(jo, carry):
        for b in range(ANBUF):
            j = jo * ANBUF + b
            nb = (b + 1) % ANBUF

            @pl.when(j < nblk)
            def _():
                @pl.when(j + 1 < nblk)
                def _():
                    @pl.when(j + 1 >= ANBUF)
                    def _():
                        wait_out(nb)
                    start_in(start + j + 1, nb)

                wait_in(b)
                _transpose_block(vbufs[b], obufs[b], 128)
                start_out(start + j, b)
        return carry

    nouter = (BLK_PER_W + 1 + ANBUF - 1) // ANBUF
    lax.fori_loop(0, nouter, step, 0)
    for b in range(ANBUF):
        wait_out(b)

    @pl.when(wid == NW - 1)
    def _():
        pltpu.sync_copy(tail_flat, obufs[1].at[pl.ds(0, TAIL_I * D)])

        kvec = [lax.iota(jnp.int32, LANES) + k * LANES
                for k in range(D // LANES)]

        def srow(i, carry):
            ii = jnp.full((LANES,), i, jnp.int32)
            for k in range(D // LANES):
                v = obufs[1][pl.ds(i * D + k * LANES, LANES)]
                rot = (kvec[k] + ii) & (D - 1)
                plsc.store_scatter(obufs[0], [ii * D + rot], v * SCALE)
            return carry

        lax.fori_loop(0, TAIL_I, srow, 0, unroll=2)
        pltpu.sync_copy(
            obufs[0].at[pl.ds(0, TAIL_I * PITCH)],
            tp_out.at[pl.ds(NBLK_FULL * 128 * PITCH, TAIL_I * PITCH)])


def _phase_b_body(xt2, tp, pe_hbm, out5, idxs, bufs, obufs, pe_v,
                  isems, gsems, ssems):
    wid = lax.axis_index("s") * NC + lax.axis_index("c")
    u0 = wid * U_PER_W

    pltpu.sync_copy(pe_hbm, pe_v)

    def start_idx(u, slot):
        pltpu.async_copy(xt2.at[u], idxs[slot], isems[slot])

    def wait_idx(slot):
        pltpu.make_async_copy(xt2.at[0], idxs[slot], isems[slot]).wait()

    def start_gather(slot):
        pltpu.async_copy(tp.at[idxs[slot]], bufs[slot], gsems[slot])

    def wait_gather(slot):
        pltpu.make_async_copy(tp.at[idxs[slot]], bufs[slot], gsems[slot]).wait()

    def start_out(u, slot):
        s = u // 8
        bb = lax.rem(u, 8)
        for cb in range(8):
            pltpu.async_copy(obufs[slot].at[cb], out5.at[s, cb, bb], ssems[slot])

    def wait_out(slot):
        for cb in range(8):
            pltpu.make_async_copy(
                obufs[slot].at[cb], out5.at[0, cb, 0], ssems[slot]).wait()

    def compute(u, slot):
        s = u // 8
        buf = bufs[slot]
        obuf = obufs[slot]
        idx_v = idxs[slot]
        bidx = [lax.iota(jnp.int32, LANES) + 16 * t for t in range(8)]
        vmod = [idx_v[pl.ds(16 * t, LANES)] & (D - 1) for t in range(8)]

        @plsc.parallel_loop(0, D, 1, unroll=4)
        def col(c):
            pev = plsc.load_gather(
                pe_v, [jnp.full((LANES,), s * D + c, jnp.int32)])
            cb = c // 8
            ci = lax.rem(c, 8)
            cc = jnp.full((LANES,), c, jnp.int32)
            for t in range(8):
                cols = (vmod[t] + cc) & (D - 1)
                v = plsc.load_gather(buf, [bidx[t], cols])
                obuf[cb, ci, pl.ds(t * LANES, LANES)] = v + pev

    start_idx(u0, 0)
    wait_idx(0)
    start_gather(0)
    start_idx(u0 + 1, 1)

    def step(jo, carry):
        for b in range(NBUF):
            j = jo * NBUF + b
            u = u0 + j
            nb = (b + 1) % NBUF

            @pl.when(j + 1 < U_PER_W)
            def _():
                wait_idx(nb)
                start_gather(nb)

            wait_gather(b)

            @pl.when(j >= NBUF)
            def _():
                wait_out(b)

            compute(u, b)
            start_out(u, b)

            @pl.when(j + 2 < U_PER_W)
            def _():
                start_idx(u + 2, b)
        return carry

    lax.fori_loop(0, U_PER_W // NBUF, step, 0)
    for b in range(NBUF):
        wait_out(b)


@jax.jit
def _run(x, table):
    mesh = plsc.VectorSubcoreMesh(core_axis_name="c", subcore_axis_name="s")

    tableT = table.T

    phase_a = pl.kernel(
        _phase_a_body,
        out_type=jax.ShapeDtypeStruct((VOCAB * PITCH,), jnp.float32),
        mesh=mesh,
        scratch_types=[
            [pltpu.VMEM((D, 128), jnp.float32) for _ in range(ANBUF)],
            [pltpu.VMEM((128 * PITCH,), jnp.float32) for _ in range(ANBUF)],
            [pltpu.SemaphoreType.DMA for _ in range(ANBUF)],
            [pltpu.SemaphoreType.DMA for _ in range(ANBUF)],
        ],
        compiler_params=pltpu.CompilerParams(use_tc_tiling_on_sc=True, needs_layout_passes=False),
    )
    tail_flat = jnp.reshape(table[NBLK_FULL * 128:, :], (TAIL_I * D,))
    tp_flat = phase_a(tableT, tail_flat)
    tp = jnp.reshape(tp_flat, (VOCAB, PITCH))

    xt2 = jnp.reshape(x.astype(jnp.int32).T, (NU, 128))
    pe_flat = jnp.asarray(_PE_FLAT)

    phase_b = pl.kernel(
        _phase_b_body,
        out_type=jax.ShapeDtypeStruct((SEQ, 8, 8, 8, 128), jnp.float32),
        mesh=mesh,
        scratch_types=[
            [pltpu.VMEM((128,), jnp.int32) for _ in range(NBUF)],
            [pltpu.VMEM((128, PITCH), jnp.float32) for _ in range(NBUF)],
            [pltpu.VMEM((8, 8, 128), jnp.float32) for _ in range(NBUF)],
            pltpu.VMEM((SEQ * D,), jnp.float32),
            [pltpu.SemaphoreType.DMA for _ in range(NBUF)],
            [pltpu.SemaphoreType.DMA for _ in range(NBUF)],
            [pltpu.SemaphoreType.DMA for _ in range(NBUF)],
        ],
        compiler_params=pltpu.CompilerParams(use_tc_tiling_on_sc=False, needs_layout_passes=False),
    )
    out5 = phase_b(xt2, tp, pe_flat)
    return jnp.transpose(out5, (2, 4, 0, 1, 3)).reshape(BATCH, SEQ, D)


def kernel(x, table):
    return _run(x, table)
